# baseline (speedup 1.0000x reference)
import functools

import jax
import jax.numpy as jnp
from jax import lax
from jax.experimental import pallas as pl
from jax.experimental.pallas import tpu as pltpu
from jax.experimental.pallas import tpu_sc as plsc

N_P = 10000
E = 320000
D = 128
NC = 2
NS = 16
NW = NC * NS
WIN = 128
N_PAD = 10240
ROWS_PER_SUB = N_PAD // NS

_sc_mesh = plsc.VectorSubcoreMesh(core_axis_name="c", subcore_axis_name="s")


KCH = 4
EC = E // KCH
NWC = EC // WIN


@functools.partial(
    pl.kernel,
    out_type=[jax.ShapeDtypeStruct((EC, D), jnp.float32),
              jax.ShapeDtypeStruct((EC, D), jnp.float32)],
    mesh=_sc_mesh,
)
def _sc_gather(xp_hbm, xc_hbm, dst_hbm, src_hbm, gp_hbm, gc_hbm):
    def body(dstb, srcb, gpb, gcb):
        pltpu.sync_copy(xp_hbm.at[dstb.at[0]], gpb)
        pltpu.sync_copy(xc_hbm.at[srcb.at[0]], gcb)

    pltpu.emit_pipeline(
        body,
        grid=(NWC,),
        in_specs=[
            pl.BlockSpec((1, WIN), lambda i: (0, i)),
            pl.BlockSpec((1, WIN), lambda i: (0, i)),
        ],
        out_specs=[
            pl.BlockSpec((WIN, D), lambda i: (i, 0)),
            pl.BlockSpec((WIN, D), lambda i: (i, 0)),
        ],
        core_axis_name=("c", "s"),
        dimension_semantics=(pltpu.PARALLEL,),
    )(dst_hbm, src_hbm, gp_hbm, gc_hbm)


EBLK = 3200


def _tc_body(gp_ref, gc_ref, amf_ref, ea_ref, w1_ref, w2_ref, out_ref):
    feat = jnp.concatenate(
        [gp_ref[...].astype(jnp.bfloat16),
         gc_ref[...].astype(jnp.bfloat16),
         amf_ref[...].astype(jnp.bfloat16)], axis=1)
    t1 = jnp.dot(feat, w1_ref[...], preferred_element_type=jnp.float32)
    ea = ea_ref[...]
    eab = [jnp.broadcast_to(ea[:, j:j + 1], (EBLK, D)) for j in range(4)]
    pre1 = (eab[0] * t1[:, 0:128] + eab[1] * t1[:, 128:256]
            + eab[2] * t1[:, 256:384] + eab[3] * t1[:, 384:512])
    m1 = pre1 * jax.nn.sigmoid(pre1)
    t2 = jnp.dot(m1.astype(jnp.bfloat16), w2_ref[...],
                 preferred_element_type=jnp.float32)
    pre2 = (eab[0] * t2[:, 0:128] + eab[1] * t2[:, 128:256]
            + eab[2] * t2[:, 256:384] + eab[3] * t2[:, 384:512])
    out_ref[...] = pre2 * jax.nn.sigmoid(pre2)


def _tc_messages(gp, gc, amf, ea, w1r, w2r):
    grid = (EC // EBLK,)
    return pl.pallas_call(
        _tc_body,
        grid=grid,
        in_specs=[
            pl.BlockSpec((EBLK, D), lambda i: (i, 0)),
            pl.BlockSpec((EBLK, D), lambda i: (i, 0)),
            pl.BlockSpec((EBLK, 16), lambda i: (i, 0)),
            pl.BlockSpec((EBLK, 4), lambda i: (i, 0)),
            pl.BlockSpec((272, 512), lambda i: (0, 0)),
            pl.BlockSpec((D, 512), lambda i: (0, 0)),
        ],
        out_specs=pl.BlockSpec((EBLK, D), lambda i: (i, 0)),
        out_shape=jax.ShapeDtypeStruct((EC, D), jnp.float32),
    )(gp, gc, amf, ea, w1r, w2r)


@functools.partial(
    pl.kernel,
    out_type=jax.ShapeDtypeStruct((NC, N_PAD, D), jnp.float32),
    mesh=_sc_mesh,
    scratch_types=[
        pltpu.VMEM_SHARED((N_PAD, D), jnp.float32),
    ],
)
def _sc_scatter(m2_0, m2_1, m2_2, m2_3, d_0, d_1, d_2, d_3, zeros_hbm,
                out_hbm, acc_sh):
    cid = lax.axis_index("c")
    sid = lax.axis_index("s")
    r0 = sid * ROWS_PER_SUB
    pltpu.sync_copy(zeros_hbm.at[pl.ds(r0, ROWS_PER_SUB)],
                    acc_sh.at[pl.ds(r0, ROWS_PER_SUB)])
    plsc.subcore_barrier()

    def body(mb, ib):
        pltpu.sync_copy(mb, acc_sh.at[ib.at[0]], add=True)

    mspec = pl.BlockSpec((WIN, D), lambda i: (i, 0))
    ispec = pl.BlockSpec((1, WIN), lambda i: (0, i))
    for m2c, dc in ((m2_0, d_0), (m2_1, d_1), (m2_2, d_2), (m2_3, d_3)):
        pltpu.emit_pipeline(
            body,
            grid=(NWC,),
            in_specs=[mspec, ispec],
            out_specs=[],
            core_axis_name=("c", "s"),
            dimension_semantics=(pltpu.PARALLEL,),
        )(m2c, dc)

    plsc.subcore_barrier()
    pltpu.sync_copy(acc_sh.at[pl.ds(r0, ROWS_PER_SUB)],
                    out_hbm.at[cid].at[pl.ds(r0, ROWS_PER_SUB)])


RBLK = 2000


def _combine_body(xp_ref, parts_ref, out_ref):
    out_ref[:, 0:D] = xp_ref[...]
    out_ref[:, D:2 * D] = parts_ref[0] + parts_ref[1]


def _tc_combine(x_p, parts):
    return pl.pallas_call(
        _combine_body,
        grid=(N_P // RBLK,),
        in_specs=[
            pl.BlockSpec((RBLK, D), lambda i: (i, 0)),
            pl.BlockSpec((NC, RBLK, D), lambda i: (0, i, 0)),
        ],
        out_specs=pl.BlockSpec((RBLK, 2 * D), lambda i: (i, 0)),
        out_shape=jax.ShapeDtypeStruct((N_P, 2 * D), jnp.float32),
    )(x_p, parts)


def kernel(x_p, x_c, edge_index, edge_attr, batch, additional_message_features,
           W1, W2):
    del batch
    src = edge_index[0].astype(jnp.int32)
    dst = edge_index[1].astype(jnp.int32)
    w1r = W1.reshape(272, 512).astype(jnp.bfloat16)
    w2r = W2.reshape(128, 512).astype(jnp.bfloat16)

    dst2 = dst.reshape(1, E)
    src2 = src.reshape(1, E)
    m2s, dsts = [], []
    for c in range(KCH):
        dc = lax.slice(dst2, (0, c * EC), (1, (c + 1) * EC))
        sc = lax.slice(src2, (0, c * EC), (1, (c + 1) * EC))
        gp, gc = _sc_gather(x_p, x_c, dc, sc)
        amf_c = lax.slice_in_dim(additional_message_features, c * EC,
                                 (c + 1) * EC, axis=0)
        ea_c = lax.slice_in_dim(edge_attr, c * EC, (c + 1) * EC, axis=0)
        m2s.append(_tc_messages(gp, gc, amf_c, ea_c, w1r, w2r))
        dsts.append(dc)
    zeros = jnp.zeros((N_PAD, D), jnp.float32)
    parts = _sc_scatter(*m2s, *dsts, zeros)
    return _tc_combine(x_p, parts)


# speedup vs baseline: 1.0004x; 1.0004x over previous
import functools

import jax
import jax.numpy as jnp
from jax import lax
from jax.experimental import pallas as pl
from jax.experimental.pallas import tpu as pltpu
from jax.experimental.pallas import tpu_sc as plsc

N_P = 10000
E = 320000
D = 128
NC = 2
NS = 16
NW = NC * NS
WIN = 128
N_PAD = 10240
ROWS_PER_SUB = N_PAD // NS

_sc_mesh = plsc.VectorSubcoreMesh(core_axis_name="c", subcore_axis_name="s")


KCH = 4
EC = E // KCH
NWC = EC // WIN


@functools.partial(
    pl.kernel,
    out_type=[jax.ShapeDtypeStruct((EC, D), jnp.float32),
              jax.ShapeDtypeStruct((EC, D), jnp.float32)],
    mesh=_sc_mesh,
)
def _sc_gather(xp_hbm, xc_hbm, dst_hbm, src_hbm, gp_hbm, gc_hbm):
    def body(dstb, srcb, gpb, gcb):
        pltpu.sync_copy(xp_hbm.at[dstb.at[0]], gpb)
        pltpu.sync_copy(xc_hbm.at[srcb.at[0]], gcb)

    pltpu.emit_pipeline(
        body,
        grid=(NWC,),
        in_specs=[
            pl.BlockSpec((1, WIN), lambda i: (0, i)),
            pl.BlockSpec((1, WIN), lambda i: (0, i)),
        ],
        out_specs=[
            pl.BlockSpec((WIN, D), lambda i: (i, 0)),
            pl.BlockSpec((WIN, D), lambda i: (i, 0)),
        ],
        core_axis_name=("c", "s"),
        dimension_semantics=(pltpu.PARALLEL,),
    )(dst_hbm, src_hbm, gp_hbm, gc_hbm)


EBLK = 3200


def _tc_body(gp_ref, gc_ref, amf_ref, ea_ref, w1_ref, w2_ref, out_ref):
    feat = jnp.concatenate(
        [gp_ref[...].astype(jnp.bfloat16),
         gc_ref[...].astype(jnp.bfloat16),
         amf_ref[...].astype(jnp.bfloat16)], axis=1)
    t1 = jnp.dot(feat, w1_ref[...], preferred_element_type=jnp.float32)
    ea = ea_ref[...]
    eab = [jnp.broadcast_to(ea[:, j:j + 1], (EBLK, D)) for j in range(4)]
    pre1 = (eab[0] * t1[:, 0:128] + eab[1] * t1[:, 128:256]
            + eab[2] * t1[:, 256:384] + eab[3] * t1[:, 384:512])
    m1 = pre1 * jax.nn.sigmoid(pre1)
    t2 = jnp.dot(m1.astype(jnp.bfloat16), w2_ref[...],
                 preferred_element_type=jnp.float32)
    pre2 = (eab[0] * t2[:, 0:128] + eab[1] * t2[:, 128:256]
            + eab[2] * t2[:, 256:384] + eab[3] * t2[:, 384:512])
    out_ref[...] = pre2 * jax.nn.sigmoid(pre2)


def _tc_messages(gp, gc, amf, ea, w1r, w2r):
    grid = (EC // EBLK,)
    return pl.pallas_call(
        _tc_body,
        grid=grid,
        in_specs=[
            pl.BlockSpec((EBLK, D), lambda i: (i, 0)),
            pl.BlockSpec((EBLK, D), lambda i: (i, 0)),
            pl.BlockSpec((EBLK, 16), lambda i: (i, 0)),
            pl.BlockSpec((EBLK, 4), lambda i: (i, 0)),
            pl.BlockSpec((272, 512), lambda i: (0, 0)),
            pl.BlockSpec((D, 512), lambda i: (0, 0)),
        ],
        out_specs=pl.BlockSpec((EBLK, D), lambda i: (i, 0)),
        out_shape=jax.ShapeDtypeStruct((EC, D), jnp.float32),
    )(gp, gc, amf, ea, w1r, w2r)


@functools.partial(
    pl.kernel,
    out_type=jax.ShapeDtypeStruct((NC, N_PAD, D), jnp.float32),
    mesh=_sc_mesh,
    scratch_types=[
        pltpu.VMEM_SHARED((N_PAD, D), jnp.float32),
    ],
)
def _sc_scatter(m2_0, m2_1, m2_2, m2_3, d_0, d_1, d_2, d_3, zeros_hbm,
                out_hbm, acc_sh):
    cid = lax.axis_index("c")
    sid = lax.axis_index("s")
    r0 = sid * ROWS_PER_SUB
    pltpu.sync_copy(zeros_hbm.at[pl.ds(r0, ROWS_PER_SUB)],
                    acc_sh.at[pl.ds(r0, ROWS_PER_SUB)])
    plsc.subcore_barrier()

    def body(mb, ib):
        pltpu.sync_copy(mb, acc_sh.at[ib.at[0]], add=True)

    mspec = pl.BlockSpec((WIN, D), lambda i: (i, 0))
    ispec = pl.BlockSpec((1, WIN), lambda i: (0, i))
    for m2c, dc in ((m2_0, d_0), (m2_1, d_1), (m2_2, d_2), (m2_3, d_3)):
        pltpu.emit_pipeline(
            body,
            grid=(NWC,),
            in_specs=[mspec, ispec],
            out_specs=[],
            core_axis_name=("c", "s"),
            dimension_semantics=(pltpu.PARALLEL,),
        )(m2c, dc)

    plsc.subcore_barrier()
    pltpu.sync_copy(acc_sh.at[pl.ds(r0, ROWS_PER_SUB)],
                    out_hbm.at[cid].at[pl.ds(r0, ROWS_PER_SUB)])


RBLK = 2000


def _combine_body(xp_ref, parts_ref, out_ref):
    out_ref[:, 0:D] = xp_ref[...]
    out_ref[:, D:2 * D] = parts_ref[0] + parts_ref[1]


def _tc_combine(x_p, parts):
    return pl.pallas_call(
        _combine_body,
        grid=(N_P // RBLK,),
        in_specs=[
            pl.BlockSpec((RBLK, D), lambda i: (i, 0)),
            pl.BlockSpec((NC, RBLK, D), lambda i: (0, i, 0)),
        ],
        out_specs=pl.BlockSpec((RBLK, 2 * D), lambda i: (i, 0)),
        out_shape=jax.ShapeDtypeStruct((N_P, 2 * D), jnp.float32),
    )(x_p, parts)


def kernel(x_p, x_c, edge_index, edge_attr, batch, additional_message_features,
           W1, W2):
    del batch
    src = edge_index[0].astype(jnp.int32)
    dst = edge_index[1].astype(jnp.int32)
    w1r = W1.reshape(272, 512).astype(jnp.bfloat16)
    w2r = W2.reshape(128, 512).astype(jnp.bfloat16)

    dst2 = dst.reshape(1, E)
    src2 = src.reshape(1, E)
    gs, dsts = [], []
    for c in range(KCH):
        dc = lax.slice(dst2, (0, c * EC), (1, (c + 1) * EC))
        sc = lax.slice(src2, (0, c * EC), (1, (c + 1) * EC))
        gs.append(_sc_gather(x_p, x_c, dc, sc))
        dsts.append(dc)
    m2s = []
    for c in range(KCH):
        gp, gc = gs[c]
        amf_c = lax.slice_in_dim(additional_message_features, c * EC,
                                 (c + 1) * EC, axis=0)
        ea_c = lax.slice_in_dim(edge_attr, c * EC, (c + 1) * EC, axis=0)
        m2s.append(_tc_messages(gp, gc, amf_c, ea_c, w1r, w2r))
    zeros = jnp.zeros((N_PAD, D), jnp.float32)
    parts = _sc_scatter(*m2s, *dsts, zeros)
    return _tc_combine(x_p, parts)


# speedup vs baseline: 1.2121x; 1.2116x over previous
import functools

import jax
import jax.numpy as jnp
from jax import lax
from jax.experimental import pallas as pl
from jax.experimental.pallas import tpu as pltpu
from jax.experimental.pallas import tpu_sc as plsc

N_P = 10000
E = 320000
D = 128
NC = 2
NS = 16
NW = NC * NS
WIN = 128
N_PAD = 10240
ROWS_PER_SUB = N_PAD // NS

_sc_mesh = plsc.VectorSubcoreMesh(core_axis_name="c", subcore_axis_name="s")


NWTOT = E // WIN
TROWS_PER_SUB = N_PAD // NS


@functools.partial(
    pl.kernel,
    out_type=[jax.ShapeDtypeStruct((E, D), jnp.float32),
              jax.ShapeDtypeStruct((E, D), jnp.float32)],
    mesh=_sc_mesh,
    scratch_types=[
        pltpu.VMEM_SHARED((N_PAD, D), jnp.float32),
    ],
)
def _sc_gather(xp_hbm, xc_hbm, dst_hbm, src_hbm, gp_hbm, gc_hbm, table_sh):
    cid = lax.axis_index("c")
    sid = lax.axis_index("s")
    r0 = sid * TROWS_PER_SUB

    @pl.when(cid == 0)
    def _():
        pltpu.sync_copy(xp_hbm.at[pl.ds(r0, TROWS_PER_SUB)],
                        table_sh.at[pl.ds(r0, TROWS_PER_SUB)])

    @pl.when(cid == 1)
    def _():
        pltpu.sync_copy(xc_hbm.at[pl.ds(r0, TROWS_PER_SUB)],
                        table_sh.at[pl.ds(r0, TROWS_PER_SUB)])

    plsc.subcore_barrier()

    def body(idxb, outb):
        pltpu.sync_copy(table_sh.at[idxb.at[0]], outb)

    ispec = [pl.BlockSpec((1, WIN), lambda i: (0, i))]
    ospec = [pl.BlockSpec((WIN, D), lambda i: (i, 0))]

    @pl.when(cid == 0)
    def _():
        pltpu.emit_pipeline(
            body, grid=(NWTOT,), in_specs=ispec, out_specs=ospec,
            core_axis_name="s", dimension_semantics=(pltpu.PARALLEL,),
        )(dst_hbm, gp_hbm)

    @pl.when(cid == 1)
    def _():
        pltpu.emit_pipeline(
            body, grid=(NWTOT,), in_specs=ispec, out_specs=ospec,
            core_axis_name="s", dimension_semantics=(pltpu.PARALLEL,),
        )(src_hbm, gc_hbm)


EBLK = 3200


def _tc_body(gp_ref, gc_ref, amf_ref, ea_ref, w1_ref, w2_ref, out_ref):
    feat = jnp.concatenate(
        [gp_ref[...].astype(jnp.bfloat16),
         gc_ref[...].astype(jnp.bfloat16),
         amf_ref[...].astype(jnp.bfloat16)], axis=1)
    t1 = jnp.dot(feat, w1_ref[...], preferred_element_type=jnp.float32)
    ea = ea_ref[...]
    eab = [jnp.broadcast_to(ea[:, j:j + 1], (EBLK, D)) for j in range(4)]
    pre1 = (eab[0] * t1[:, 0:128] + eab[1] * t1[:, 128:256]
            + eab[2] * t1[:, 256:384] + eab[3] * t1[:, 384:512])
    m1 = pre1 * jax.nn.sigmoid(pre1)
    t2 = jnp.dot(m1.astype(jnp.bfloat16), w2_ref[...],
                 preferred_element_type=jnp.float32)
    pre2 = (eab[0] * t2[:, 0:128] + eab[1] * t2[:, 128:256]
            + eab[2] * t2[:, 256:384] + eab[3] * t2[:, 384:512])
    out_ref[...] = pre2 * jax.nn.sigmoid(pre2)


def _tc_messages(gp, gc, amf, ea, w1r, w2r):
    grid = (E // EBLK,)
    return pl.pallas_call(
        _tc_body,
        grid=grid,
        in_specs=[
            pl.BlockSpec((EBLK, D), lambda i: (i, 0)),
            pl.BlockSpec((EBLK, D), lambda i: (i, 0)),
            pl.BlockSpec((EBLK, 16), lambda i: (i, 0)),
            pl.BlockSpec((EBLK, 4), lambda i: (i, 0)),
            pl.BlockSpec((272, 512), lambda i: (0, 0)),
            pl.BlockSpec((D, 512), lambda i: (0, 0)),
        ],
        out_specs=pl.BlockSpec((EBLK, D), lambda i: (i, 0)),
        out_shape=jax.ShapeDtypeStruct((E, D), jnp.float32),
    )(gp, gc, amf, ea, w1r, w2r)


@functools.partial(
    pl.kernel,
    out_type=jax.ShapeDtypeStruct((NC, N_PAD, D), jnp.float32),
    mesh=_sc_mesh,
    scratch_types=[
        pltpu.VMEM_SHARED((N_PAD, D), jnp.float32),
    ],
)
def _sc_scatter(m2_hbm, dst_hbm, zeros_hbm, out_hbm, acc_sh):
    cid = lax.axis_index("c")
    sid = lax.axis_index("s")
    r0 = sid * ROWS_PER_SUB
    pltpu.sync_copy(zeros_hbm.at[pl.ds(r0, ROWS_PER_SUB)],
                    acc_sh.at[pl.ds(r0, ROWS_PER_SUB)])
    plsc.subcore_barrier()

    def body(mb, ib):
        pltpu.sync_copy(mb, acc_sh.at[ib.at[0]], add=True)

    pltpu.emit_pipeline(
        body,
        grid=(NWTOT,),
        in_specs=[
            pl.BlockSpec((WIN, D), lambda i: (i, 0)),
            pl.BlockSpec((1, WIN), lambda i: (0, i)),
        ],
        out_specs=[],
        core_axis_name=("c", "s"),
        dimension_semantics=(pltpu.PARALLEL,),
    )(m2_hbm, dst_hbm)

    plsc.subcore_barrier()
    pltpu.sync_copy(acc_sh.at[pl.ds(r0, ROWS_PER_SUB)],
                    out_hbm.at[cid].at[pl.ds(r0, ROWS_PER_SUB)])


RBLK = 2000


def _combine_body(xp_ref, parts_ref, out_ref):
    out_ref[:, 0:D] = xp_ref[...]
    out_ref[:, D:2 * D] = parts_ref[0] + parts_ref[1]


def _tc_combine(x_p, parts):
    return pl.pallas_call(
        _combine_body,
        grid=(N_P // RBLK,),
        in_specs=[
            pl.BlockSpec((RBLK, D), lambda i: (i, 0)),
            pl.BlockSpec((NC, RBLK, D), lambda i: (0, i, 0)),
        ],
        out_specs=pl.BlockSpec((RBLK, 2 * D), lambda i: (i, 0)),
        out_shape=jax.ShapeDtypeStruct((N_P, 2 * D), jnp.float32),
    )(x_p, parts)


def kernel(x_p, x_c, edge_index, edge_attr, batch, additional_message_features,
           W1, W2):
    del batch
    src = edge_index[0].astype(jnp.int32)
    dst = edge_index[1].astype(jnp.int32)
    w1r = W1.reshape(272, 512).astype(jnp.bfloat16)
    w2r = W2.reshape(128, 512).astype(jnp.bfloat16)

    dst2 = dst.reshape(1, E)
    src2 = src.reshape(1, E)
    xp_b = jnp.zeros((N_PAD, D), jnp.float32).at[:N_P].set(x_p)
    xc_b = jnp.zeros((N_PAD, D), jnp.float32).at[:N_P].set(x_c[:N_P])
    gp, gc = _sc_gather(xp_b, xc_b, dst2, src2)
    m2 = _tc_messages(gp, gc, additional_message_features, edge_attr,
                      w1r, w2r)
    zeros = jnp.zeros((N_PAD, D), jnp.float32)
    parts = _sc_scatter(m2, dst2, zeros)
    return _tc_combine(x_p, parts)


# speedup vs baseline: 1.2563x; 1.0365x over previous
import functools

import jax
import jax.numpy as jnp
from jax import lax
from jax.experimental import pallas as pl
from jax.experimental.pallas import tpu as pltpu
from jax.experimental.pallas import tpu_sc as plsc

N_P = 10000
E = 320000
D = 128
NC = 2
NS = 16
NW = NC * NS
WIN = 128
N_PAD = 10240
ROWS_PER_SUB = N_PAD // NS

_sc_mesh = plsc.VectorSubcoreMesh(core_axis_name="c", subcore_axis_name="s")


NWTOT = E // WIN
TROWS_PER_SUB = N_PAD // NS


@functools.partial(
    pl.kernel,
    out_type=[jax.ShapeDtypeStruct((E, D), jnp.float32),
              jax.ShapeDtypeStruct((E, D), jnp.float32)],
    mesh=_sc_mesh,
    scratch_types=[
        pltpu.VMEM_SHARED((N_PAD, D), jnp.float32),
    ],
)
def _sc_gather(xp_hbm, xc_hbm, dst_hbm, src_hbm, gp_hbm, gc_hbm, table_sh):
    cid = lax.axis_index("c")
    sid = lax.axis_index("s")
    r0 = sid * TROWS_PER_SUB

    @pl.when(cid == 0)
    def _():
        pltpu.sync_copy(xp_hbm.at[pl.ds(r0, TROWS_PER_SUB)],
                        table_sh.at[pl.ds(r0, TROWS_PER_SUB)])

    @pl.when(cid == 1)
    def _():
        pltpu.sync_copy(xc_hbm.at[pl.ds(r0, TROWS_PER_SUB)],
                        table_sh.at[pl.ds(r0, TROWS_PER_SUB)])

    plsc.subcore_barrier()

    def body(idxb, outb):
        pltpu.sync_copy(table_sh.at[idxb.at[0]], outb)

    ispec = [pl.BlockSpec((1, WIN), lambda i: (0, i))]
    ospec = [pl.BlockSpec((WIN, D), lambda i: (i, 0))]

    @pl.when(cid == 0)
    def _():
        pltpu.emit_pipeline(
            body, grid=(NWTOT,), in_specs=ispec, out_specs=ospec,
            core_axis_name="s", dimension_semantics=(pltpu.PARALLEL,),
        )(dst_hbm, gp_hbm)

    @pl.when(cid == 1)
    def _():
        pltpu.emit_pipeline(
            body, grid=(NWTOT,), in_specs=ispec, out_specs=ospec,
            core_axis_name="s", dimension_semantics=(pltpu.PARALLEL,),
        )(src_hbm, gc_hbm)


EBLK = 6400


def _tc_body(gp_ref, gc_ref, amf_ref, ea_ref, w1_ref, w2_ref, out_ref):
    feat = jnp.concatenate(
        [gp_ref[...].astype(jnp.bfloat16),
         gc_ref[...].astype(jnp.bfloat16),
         amf_ref[...].astype(jnp.bfloat16)], axis=1)
    t1 = jnp.dot(feat, w1_ref[...], preferred_element_type=jnp.float32)
    ea = ea_ref[...]
    eab = [jnp.broadcast_to(ea[:, j:j + 1], (EBLK, D)) for j in range(4)]
    pre1 = (eab[0] * t1[:, 0:128] + eab[1] * t1[:, 128:256]
            + eab[2] * t1[:, 256:384] + eab[3] * t1[:, 384:512])
    m1 = pre1 * jax.nn.sigmoid(pre1)
    t2 = jnp.dot(m1.astype(jnp.bfloat16), w2_ref[...],
                 preferred_element_type=jnp.float32)
    pre2 = (eab[0] * t2[:, 0:128] + eab[1] * t2[:, 128:256]
            + eab[2] * t2[:, 256:384] + eab[3] * t2[:, 384:512])
    out_ref[...] = pre2 * jax.nn.sigmoid(pre2)


def _tc_messages(gp, gc, amf, ea, w1r, w2r):
    grid = (E // EBLK,)
    return pl.pallas_call(
        _tc_body,
        grid=grid,
        in_specs=[
            pl.BlockSpec((EBLK, D), lambda i: (i, 0)),
            pl.BlockSpec((EBLK, D), lambda i: (i, 0)),
            pl.BlockSpec((EBLK, 16), lambda i: (i, 0)),
            pl.BlockSpec((EBLK, 4), lambda i: (i, 0)),
            pl.BlockSpec((272, 512), lambda i: (0, 0)),
            pl.BlockSpec((D, 512), lambda i: (0, 0)),
        ],
        out_specs=pl.BlockSpec((EBLK, D), lambda i: (i, 0)),
        out_shape=jax.ShapeDtypeStruct((E, D), jnp.float32),
    )(gp, gc, amf, ea, w1r, w2r)


@functools.partial(
    pl.kernel,
    out_type=jax.ShapeDtypeStruct((NC, N_PAD, D), jnp.float32),
    mesh=_sc_mesh,
    scratch_types=[
        pltpu.VMEM_SHARED((N_PAD, D), jnp.float32),
    ],
)
def _sc_scatter(m2_hbm, dst_hbm, zeros_hbm, out_hbm, acc_sh):
    cid = lax.axis_index("c")
    sid = lax.axis_index("s")
    r0 = sid * ROWS_PER_SUB
    pltpu.sync_copy(zeros_hbm.at[pl.ds(r0, ROWS_PER_SUB)],
                    acc_sh.at[pl.ds(r0, ROWS_PER_SUB)])
    plsc.subcore_barrier()

    def body(mb, ib):
        pltpu.sync_copy(mb, acc_sh.at[ib.at[0]], add=True)

    pltpu.emit_pipeline(
        body,
        grid=(NWTOT,),
        in_specs=[
            pl.BlockSpec((WIN, D), lambda i: (i, 0)),
            pl.BlockSpec((1, WIN), lambda i: (0, i)),
        ],
        out_specs=[],
        core_axis_name=("c", "s"),
        dimension_semantics=(pltpu.PARALLEL,),
    )(m2_hbm, dst_hbm)

    plsc.subcore_barrier()
    pltpu.sync_copy(acc_sh.at[pl.ds(r0, ROWS_PER_SUB)],
                    out_hbm.at[cid].at[pl.ds(r0, ROWS_PER_SUB)])


RBLK = 2000


def _combine_body(xp_ref, parts_ref, out_ref):
    out_ref[:, 0:D] = xp_ref[...]
    out_ref[:, D:2 * D] = parts_ref[0] + parts_ref[1]


def _tc_combine(x_p, parts):
    return pl.pallas_call(
        _combine_body,
        grid=(N_P // RBLK,),
        in_specs=[
            pl.BlockSpec((RBLK, D), lambda i: (i, 0)),
            pl.BlockSpec((NC, RBLK, D), lambda i: (0, i, 0)),
        ],
        out_specs=pl.BlockSpec((RBLK, 2 * D), lambda i: (i, 0)),
        out_shape=jax.ShapeDtypeStruct((N_P, 2 * D), jnp.float32),
    )(x_p, parts)


def kernel(x_p, x_c, edge_index, edge_attr, batch, additional_message_features,
           W1, W2):
    del batch
    src = edge_index[0].astype(jnp.int32)
    dst = edge_index[1].astype(jnp.int32)
    w1r = W1.reshape(272, 512).astype(jnp.bfloat16)
    w2r = W2.reshape(128, 512).astype(jnp.bfloat16)

    dst2 = dst.reshape(1, E)
    src2 = src.reshape(1, E)
    xp_b = jnp.zeros((N_PAD, D), jnp.float32).at[:N_P].set(x_p)
    xc_b = jnp.zeros((N_PAD, D), jnp.float32).at[:N_P].set(x_c[:N_P])
    gp, gc = _sc_gather(xp_b, xc_b, dst2, src2)
    m2 = _tc_messages(gp, gc, additional_message_features, edge_attr,
                      w1r, w2r)
    zeros = jnp.zeros((N_PAD, D), jnp.float32)
    parts = _sc_scatter(m2, dst2, zeros)
    return _tc_combine(x_p, parts)
